# Initial kernel scaffold; baseline (speedup 1.0000x reference)
import jax
import jax.numpy as jnp
from jax.experimental import pallas as pl


def kernel(points, W0, b0, g0, be0, W1, b1, g1, be1):
    raise NotImplementedError("write your pallas kernel here")


# speedup vs baseline: 3.5461x; 3.5461x over previous
import functools

import jax
import jax.numpy as jnp
from jax import lax
from jax.experimental import pallas as pl
from jax.experimental.pallas import tpu as pltpu
from jax.experimental.pallas import tpu_sc as plsc

NPTS = 10000
KNN_K = 16
BN_EPS = 1e-5

SC_CORES = 2
SC_SUBCORES = 16
NWORK = SC_CORES * SC_SUBCORES
NPAD = 10240
BPW = NPAD // NWORK
GP = 8
NCHUNK = BPW // GP

QCHUNK = 40
ECHUNK = 128


def _knn_body(q_ref, pt_ref, idx_ref, dist_ref):
    qx = q_ref[:, 0:1]
    qy = q_ref[:, 1:2]
    qz = q_ref[:, 2:3]
    px = pt_ref[0:1, :]
    py = pt_ref[1:2, :]
    pz = pt_ref[2:3, :]
    dx = qx - px
    dy = qy - py
    dz = qz - pz
    dist_ref[...] = dx * dx + dy * dy + dz * dz
    col = lax.broadcasted_iota(jnp.int32, (QCHUNK, NPTS), 1)
    col16 = lax.broadcasted_iota(jnp.int32, (QCHUNK, KNN_K), 1)
    big_i = jnp.int32(NPTS)
    inf = jnp.float32(jnp.inf)

    def pass_body(k, idx_acc):
        d = dist_ref[...]
        m = jnp.min(d, axis=1, keepdims=True)
        cand = jnp.where(d == m, col, big_i)
        amin = jnp.min(cand, axis=1, keepdims=True)
        idx_acc = jnp.where(col16 == k, amin, idx_acc)
        dist_ref[...] = jnp.where(col == amin, inf, d)
        return idx_acc

    idx0 = jnp.zeros((QCHUNK, KNN_K), jnp.int32)
    idx_ref[...] = lax.fori_loop(0, KNN_K, pass_body, idx0)


def _knn(pos, pos_t):
    return pl.pallas_call(
        _knn_body,
        grid=(NPTS // QCHUNK,),
        in_specs=[
            pl.BlockSpec((QCHUNK, 3), lambda i: (i, 0)),
            pl.BlockSpec((3, NPTS), lambda i: (0, 0)),
        ],
        out_specs=pl.BlockSpec((QCHUNK, KNN_K), lambda i: (i, 0)),
        out_shape=jax.ShapeDtypeStruct((NPTS, KNN_K), jnp.int32),
        scratch_shapes=[pltpu.VMEM((QCHUNK, NPTS), jnp.float32)],
    )(pos, pos_t)


def _make_sc_gather(c):
    mesh = plsc.VectorSubcoreMesh(core_axis_name="c", subcore_axis_name="s")

    @functools.partial(
        pl.kernel,
        mesh=mesh,
        compiler_params=pltpu.CompilerParams(use_tc_tiling_on_sc=False),
        out_type=jax.ShapeDtypeStruct((NPAD * KNN_K, c), jnp.float32),
        scratch_types=[
            pltpu.VMEM((GP * KNN_K,), jnp.int32),
            pltpu.VMEM((GP * KNN_K, c), jnp.float32),
            pltpu.SemaphoreType.DMA,
        ],
    )
    def kern(x_hbm, idxf_hbm, out_hbm, idx_v, rows_v, sem):
        wid = lax.axis_index("s") * SC_CORES + lax.axis_index("c")

        def chunk_body(ci, _):
            base_e = (wid * BPW + ci * GP) * KNN_K
            pltpu.sync_copy(idxf_hbm.at[pl.ds(base_e, GP * KNN_K)], idx_v)
            pltpu.async_copy(x_hbm.at[idx_v], rows_v, sem).wait()
            pltpu.sync_copy(rows_v, out_hbm.at[pl.ds(base_e, GP * KNN_K)])
            return ()

        lax.fori_loop(0, NCHUNK, chunk_body, ())

    return kern


def _econv_body(x_ref, xg_ref, wa_ref, wb_ref, b_ref,
                mx_ref, sh_ref, sh2_ref):
    i = pl.program_id(0)
    c = x_ref.shape[1]
    d = wa_ref.shape[1]
    xi = x_ref[...]
    xi3 = jnp.broadcast_to(xi[:, None, :], (ECHUNK, KNN_K, c))
    dvec = xg_ref[...] - xi3.reshape(ECHUNK * KNN_K, c)
    p = jnp.dot(xi, wa_ref[...], preferred_element_type=jnp.float32)
    e = jnp.dot(dvec, wb_ref[...], preferred_element_type=jnp.float32)
    h = (e.reshape(ECHUNK, KNN_K, d) + p[:, None, :]) + b_ref[...][None]
    mx_ref[...] = jnp.max(h, axis=1)
    node = i * ECHUNK + lax.broadcasted_iota(jnp.int32, (ECHUNK, KNN_K, d), 0)
    hv = jnp.where(node < NPTS, h, 0.0)
    psum = jnp.sum(hv, axis=(0, 1), keepdims=False)[None]
    psum2 = jnp.sum(hv * hv, axis=(0, 1), keepdims=False)[None]

    @pl.when(i == 0)
    def _():
        sh_ref[...] = jnp.zeros_like(sh_ref)
        sh2_ref[...] = jnp.zeros_like(sh2_ref)

    sh_ref[...] += psum
    sh2_ref[...] += psum2


def _econv(x_pad, xgather, wa, wb, b):
    c = x_pad.shape[1]
    d = wa.shape[1]
    return pl.pallas_call(
        _econv_body,
        grid=(NPAD // ECHUNK,),
        in_specs=[
            pl.BlockSpec((ECHUNK, c), lambda i: (i, 0)),
            pl.BlockSpec((ECHUNK * KNN_K, c), lambda i: (i, 0)),
            pl.BlockSpec((c, d), lambda i: (0, 0)),
            pl.BlockSpec((c, d), lambda i: (0, 0)),
            pl.BlockSpec((1, d), lambda i: (0, 0)),
        ],
        out_specs=[
            pl.BlockSpec((ECHUNK, d), lambda i: (i, 0)),
            pl.BlockSpec((1, d), lambda i: (0, 0)),
            pl.BlockSpec((1, d), lambda i: (0, 0)),
        ],
        out_shape=[
            jax.ShapeDtypeStruct((NPAD, d), jnp.float32),
            jax.ShapeDtypeStruct((1, d), jnp.float32),
            jax.ShapeDtypeStruct((1, d), jnp.float32),
        ],
    )(x_pad, xgather, wa, wb, b)


def _combine_body(mx_ref, sh_ref, sh2_ref, g_ref, be_ref, o_ref):
    cnt = jnp.float32(NPTS * KNN_K)
    mu = sh_ref[...] / cnt
    ex2 = sh2_ref[...] / cnt
    var = ex2 - mu * mu
    scale = g_ref[...] * lax.rsqrt(var + BN_EPS)
    o_ref[...] = jnp.maximum((mx_ref[...] - mu) * scale + be_ref[...], 0.0)


def _combine(maxh, sh, sh2, g, be, rows_chunk=2000):
    n, d = maxh.shape
    return pl.pallas_call(
        _combine_body,
        grid=(n // rows_chunk,),
        in_specs=[
            pl.BlockSpec((rows_chunk, d), lambda i: (i, 0)),
            pl.BlockSpec((1, d), lambda i: (0, 0)),
            pl.BlockSpec((1, d), lambda i: (0, 0)),
            pl.BlockSpec((1, d), lambda i: (0, 0)),
            pl.BlockSpec((1, d), lambda i: (0, 0)),
        ],
        out_specs=pl.BlockSpec((rows_chunk, d), lambda i: (i, 0)),
        out_shape=jax.ShapeDtypeStruct((n, d), jnp.float32),
    )(maxh, sh, sh2, g, be)


def _edge_conv_layer(x, idx_flat, w, b, g, be, sc_kern, cpad):
    c = x.shape[1]
    d = w.shape[1]
    wa = jnp.zeros((cpad, d), jnp.float32).at[:c].set(w[:c])
    wb = jnp.zeros((cpad, d), jnp.float32).at[:c].set(w[c:])
    x_pad = jnp.zeros((NPAD, cpad), jnp.float32).at[:NPTS, :c].set(x)
    xgather = sc_kern(x_pad, idx_flat)
    maxh, sh, sh2 = _econv(x_pad, xgather, wa, wb, b[None])
    return _combine(maxh[:NPTS], sh, sh2, g[None], be[None])


def kernel(points, W0, b0, g0, be0, W1, b1, g1, be1):
    pos = points[:, 1:4]
    pos_t = pos.T
    idx = _knn(pos, pos_t)
    idx_pad = jnp.zeros((NPAD, KNN_K), jnp.int32).at[:NPTS].set(idx)
    idx_flat = idx_pad.reshape(-1)

    feat = points[:, 1:]
    sc16 = _make_sc_gather(16)
    sc32 = _make_sc_gather(32)
    feat = _edge_conv_layer(feat, idx_flat, W0, b0, g0, be0, sc16, 16)
    feat = _edge_conv_layer(feat, idx_flat, W1, b1, g1, be1, sc32, 32)
    return feat


# speedup vs baseline: 5.6137x; 1.5831x over previous
import functools

import jax
import jax.numpy as jnp
from jax import lax
from jax.experimental import pallas as pl
from jax.experimental.pallas import tpu as pltpu
from jax.experimental.pallas import tpu_sc as plsc

NPTS = 10000
KNN_K = 16
BN_EPS = 1e-5

SC_CORES = 2
SC_SUBCORES = 16
NWORK = SC_CORES * SC_SUBCORES
NPAD = 10240
BPW = NPAD // NWORK
GP = 8
NCHUNK = BPW // GP

ECHUNK = 128

QT = 128
CP = 10240
BS = 128
NB = CP // BS
SGS = 1024
NSG = CP // SGS
HEADS = 6
PAD_COORD = 1e9


def _knn_dist_slice(p_ref, qt_ref, sg):
    sl = pl.ds(sg * SGS, SGS)
    px = p_ref[sl, 0:1]
    py = p_ref[sl, 1:2]
    pz = p_ref[sl, 2:3]
    qx = qt_ref[0:1, :]
    qy = qt_ref[1:2, :]
    qz = qt_ref[2:3, :]
    dx = px - qx
    dy = py - qy
    dz = pz - qz
    return dx * dx + dy * dy + dz * dz


def _knn_body(p_ref, qt_ref, idx_ref, d_ref, hv_ref, hi_ref):
    inf = jnp.float32(jnp.inf)
    big_i = jnp.int32(1 << 30)

    def build_sg(sg, _):
        d_ref[pl.ds(sg * SGS, SGS), :] = _knn_dist_slice(p_ref, qt_ref, sg)
        return ()

    lax.fori_loop(0, NSG, build_sg, ())

    def head_round(r, _):
        def head_sg(sg, _):
            sl = pl.ds(sg * SGS, SGS)
            dsg = d_ref[sl, :]
            v3 = dsg.reshape(SGS // BS, BS, QT)
            bm = jnp.min(v3, axis=1)
            bmb = jnp.broadcast_to(bm[:, None, :], (SGS // BS, BS, QT))
            bmb = bmb.reshape(SGS, QT)
            gcol = sg * SGS + lax.broadcasted_iota(jnp.int32, (SGS, QT), 0)
            cand = jnp.where(dsg == bmb, gcol, big_i)
            pos = jnp.min(cand.reshape(SGS // BS, BS, QT), axis=1)
            posb = jnp.broadcast_to(pos[:, None, :], (SGS // BS, BS, QT))
            posb = posb.reshape(SGS, QT)
            d_ref[sl, :] = jnp.where(gcol == posb, inf, dsg)
            hsl = pl.ds(r * NB + sg * (SGS // BS), SGS // BS)
            hv_ref[hsl, :] = bm
            hi_ref[hsl, :] = pos
            return ()

        lax.fori_loop(0, NSG, head_sg, ())
        return ()

    lax.fori_loop(0, HEADS, head_round, ())

    row16 = lax.broadcasted_iota(jnp.int32, (KNN_K, QT), 0)
    hrank = lax.broadcasted_iota(jnp.int32, (HEADS * NB, QT), 0) // NB

    def sel_pass(k, carry):
        idx_acc, exfl = carry
        hv = hv_ref[...]
        hi = hi_ref[...]
        m = jnp.min(hv, axis=0, keepdims=True)
        sel = hv == m
        cand = jnp.where(sel, hi, big_i)
        amin = jnp.min(cand, axis=0, keepdims=True)
        taken = hi == amin
        idx_acc = jnp.where(row16 == k, jnp.broadcast_to(amin, (KNN_K, QT)),
                            idx_acc)
        selr = jnp.max(jnp.where(taken & sel, hrank, -1), axis=0,
                       keepdims=True)
        exfl = jnp.maximum(exfl, selr)
        hv_ref[...] = jnp.where(taken, inf, hv)
        return idx_acc, exfl

    idx0 = jnp.zeros((KNN_K, QT), jnp.int32)
    ex0 = jnp.full((1, QT), -1, jnp.int32)
    idx_acc, exfl = lax.fori_loop(0, KNN_K, sel_pass, (idx0, ex0))
    idx_ref[...] = idx_acc

    @pl.when(jnp.max(exfl) >= HEADS - 1)
    def _():
        def rebuild_sg(sg, _):
            d_ref[pl.ds(sg * SGS, SGS), :] = _knn_dist_slice(p_ref, qt_ref, sg)
            return ()

        lax.fori_loop(0, NSG, rebuild_sg, ())
        gcol = lax.broadcasted_iota(jnp.int32, (CP, QT), 0)

        def fb_pass(k, idx_acc):
            d = d_ref[...]
            m = jnp.min(d, axis=0, keepdims=True)
            cand = jnp.where(d == m, gcol, big_i)
            amin = jnp.min(cand, axis=0, keepdims=True)
            idx_acc = jnp.where(row16 == k,
                                jnp.broadcast_to(amin, (KNN_K, QT)), idx_acc)
            d_ref[...] = jnp.where(gcol == amin, inf, d)
            return idx_acc

        idx_fb = lax.fori_loop(0, KNN_K, fb_pass, idx0)
        idx_ref[...] = idx_fb


def _knn(pos_pad, pos_t_pad):
    return pl.pallas_call(
        _knn_body,
        grid=(CP // QT,),
        in_specs=[
            pl.BlockSpec((CP, 3), lambda i: (0, 0)),
            pl.BlockSpec((3, QT), lambda i: (0, i)),
        ],
        out_specs=pl.BlockSpec((KNN_K, QT), lambda i: (0, i)),
        out_shape=jax.ShapeDtypeStruct((KNN_K, CP), jnp.int32),
        scratch_shapes=[
            pltpu.VMEM((CP, QT), jnp.float32),
            pltpu.VMEM((HEADS * NB, QT), jnp.float32),
            pltpu.VMEM((HEADS * NB, QT), jnp.int32),
        ],
    )(pos_pad, pos_t_pad)


def _make_sc_gather(c):
    mesh = plsc.VectorSubcoreMesh(core_axis_name="c", subcore_axis_name="s")

    @functools.partial(
        pl.kernel,
        mesh=mesh,
        compiler_params=pltpu.CompilerParams(use_tc_tiling_on_sc=False),
        out_type=jax.ShapeDtypeStruct((NPAD * KNN_K, c), jnp.float32),
        scratch_types=[
            pltpu.VMEM((GP * KNN_K,), jnp.int32),
            pltpu.VMEM((GP * KNN_K, c), jnp.float32),
            pltpu.SemaphoreType.DMA,
        ],
    )
    def kern(x_hbm, idxf_hbm, out_hbm, idx_v, rows_v, sem):
        wid = lax.axis_index("s") * SC_CORES + lax.axis_index("c")

        def chunk_body(ci, _):
            base_e = (wid * BPW + ci * GP) * KNN_K
            pltpu.sync_copy(idxf_hbm.at[pl.ds(base_e, GP * KNN_K)], idx_v)
            pltpu.async_copy(x_hbm.at[idx_v], rows_v, sem).wait()
            pltpu.sync_copy(rows_v, out_hbm.at[pl.ds(base_e, GP * KNN_K)])
            return ()

        lax.fori_loop(0, NCHUNK, chunk_body, ())

    return kern


def _econv_body(x_ref, xg_ref, wa_ref, wb_ref, b_ref,
                mx_ref, sh_ref, sh2_ref):
    i = pl.program_id(0)
    c = x_ref.shape[1]
    d = wa_ref.shape[1]
    xi = x_ref[...]
    xi3 = jnp.broadcast_to(xi[:, None, :], (ECHUNK, KNN_K, c))
    dvec = xg_ref[...] - xi3.reshape(ECHUNK * KNN_K, c)
    p = jnp.dot(xi, wa_ref[...], preferred_element_type=jnp.float32)
    e = jnp.dot(dvec, wb_ref[...], preferred_element_type=jnp.float32)
    h = (e.reshape(ECHUNK, KNN_K, d) + p[:, None, :]) + b_ref[...][None]
    mx_ref[...] = jnp.max(h, axis=1)
    node = i * ECHUNK + lax.broadcasted_iota(jnp.int32, (ECHUNK, KNN_K, d), 0)
    hv = jnp.where(node < NPTS, h, 0.0)
    psum = jnp.sum(hv, axis=(0, 1), keepdims=False)[None]
    psum2 = jnp.sum(hv * hv, axis=(0, 1), keepdims=False)[None]

    @pl.when(i == 0)
    def _():
        sh_ref[...] = jnp.zeros_like(sh_ref)
        sh2_ref[...] = jnp.zeros_like(sh2_ref)

    sh_ref[...] += psum
    sh2_ref[...] += psum2


def _econv(x_pad, xgather, wa, wb, b):
    c = x_pad.shape[1]
    d = wa.shape[1]
    return pl.pallas_call(
        _econv_body,
        grid=(NPAD // ECHUNK,),
        in_specs=[
            pl.BlockSpec((ECHUNK, c), lambda i: (i, 0)),
            pl.BlockSpec((ECHUNK * KNN_K, c), lambda i: (i, 0)),
            pl.BlockSpec((c, d), lambda i: (0, 0)),
            pl.BlockSpec((c, d), lambda i: (0, 0)),
            pl.BlockSpec((1, d), lambda i: (0, 0)),
        ],
        out_specs=[
            pl.BlockSpec((ECHUNK, d), lambda i: (i, 0)),
            pl.BlockSpec((1, d), lambda i: (0, 0)),
            pl.BlockSpec((1, d), lambda i: (0, 0)),
        ],
        out_shape=[
            jax.ShapeDtypeStruct((NPAD, d), jnp.float32),
            jax.ShapeDtypeStruct((1, d), jnp.float32),
            jax.ShapeDtypeStruct((1, d), jnp.float32),
        ],
    )(x_pad, xgather, wa, wb, b)


def _combine_body(mx_ref, sh_ref, sh2_ref, g_ref, be_ref, o_ref):
    cnt = jnp.float32(NPTS * KNN_K)
    mu = sh_ref[...] / cnt
    ex2 = sh2_ref[...] / cnt
    var = ex2 - mu * mu
    scale = g_ref[...] * lax.rsqrt(var + BN_EPS)
    o_ref[...] = jnp.maximum((mx_ref[...] - mu) * scale + be_ref[...], 0.0)


def _combine(maxh, sh, sh2, g, be, rows_chunk=2000):
    n, d = maxh.shape
    return pl.pallas_call(
        _combine_body,
        grid=(n // rows_chunk,),
        in_specs=[
            pl.BlockSpec((rows_chunk, d), lambda i: (i, 0)),
            pl.BlockSpec((1, d), lambda i: (0, 0)),
            pl.BlockSpec((1, d), lambda i: (0, 0)),
            pl.BlockSpec((1, d), lambda i: (0, 0)),
            pl.BlockSpec((1, d), lambda i: (0, 0)),
        ],
        out_specs=pl.BlockSpec((rows_chunk, d), lambda i: (i, 0)),
        out_shape=jax.ShapeDtypeStruct((n, d), jnp.float32),
    )(maxh, sh, sh2, g, be)


def _edge_conv_layer(x, idx_flat, w, b, g, be, sc_kern, cpad):
    c = x.shape[1]
    d = w.shape[1]
    wa = jnp.zeros((cpad, d), jnp.float32).at[:c].set(w[:c])
    wb = jnp.zeros((cpad, d), jnp.float32).at[:c].set(w[c:])
    x_pad = jnp.zeros((NPAD, cpad), jnp.float32).at[:NPTS, :c].set(x)
    xgather = sc_kern(x_pad, idx_flat)
    maxh, sh, sh2 = _econv(x_pad, xgather, wa, wb, b[None])
    return _combine(maxh[:NPTS], sh, sh2, g[None], be[None])


def kernel(points, W0, b0, g0, be0, W1, b1, g1, be1):
    pos = points[:, 1:4]
    pos_pad = jnp.full((CP, 3), PAD_COORD, jnp.float32).at[:NPTS].set(pos)
    idx_t = _knn(pos_pad, pos_pad.T)
    idx = idx_t[:, :NPTS].T
    idx_pad = jnp.zeros((NPAD, KNN_K), jnp.int32).at[:NPTS].set(idx)
    idx_flat = idx_pad.reshape(-1)

    feat = points[:, 1:]
    sc16 = _make_sc_gather(16)
    sc32 = _make_sc_gather(32)
    feat = _edge_conv_layer(feat, idx_flat, W0, b0, g0, be0, sc16, 16)
    feat = _edge_conv_layer(feat, idx_flat, W1, b1, g1, be1, sc32, 32)
    return feat


# speedup vs baseline: 5.7731x; 1.0284x over previous
import functools

import jax
import jax.numpy as jnp
from jax import lax
from jax.experimental import pallas as pl
from jax.experimental.pallas import tpu as pltpu
from jax.experimental.pallas import tpu_sc as plsc

NPTS = 10000
KNN_K = 16
BN_EPS = 1e-5

SC_CORES = 2
SC_SUBCORES = 16
NWORK = SC_CORES * SC_SUBCORES
NPAD = 10240
BPW = NPAD // NWORK
GP = 8
NCHUNK = BPW // GP

ECHUNK = 128

QT = 128
CP = 10240
BS = 128
NB = CP // BS
SGS = 1024
NSG = CP // SGS
HEADS = 6
PAD_COORD = 1e9


def _knn_dist_slice(p_ref, qt_ref, sg):
    sl = pl.ds(sg * SGS, SGS)
    px = p_ref[sl, 0:1]
    py = p_ref[sl, 1:2]
    pz = p_ref[sl, 2:3]
    qx = qt_ref[0:1, :]
    qy = qt_ref[1:2, :]
    qz = qt_ref[2:3, :]
    dx = px - qx
    dy = py - qy
    dz = pz - qz
    return dx * dx + dy * dy + dz * dz


def _knn_body(p_ref, qt_ref, idx_ref, d_ref, hv_ref, hi_ref):
    inf = jnp.float32(jnp.inf)
    big_i = jnp.int32(1 << 30)

    def build_sg(sg, _):
        d_ref[pl.ds(sg * SGS, SGS), :] = _knn_dist_slice(p_ref, qt_ref, sg)
        return ()

    lax.fori_loop(0, NSG, build_sg, ())

    def head_round(r, _):
        def head_sg(sg, _):
            sl = pl.ds(sg * SGS, SGS)
            dsg = d_ref[sl, :]
            v3 = dsg.reshape(SGS // BS, BS, QT)
            bm = jnp.min(v3, axis=1)
            bmb = jnp.broadcast_to(bm[:, None, :], (SGS // BS, BS, QT))
            bmb = bmb.reshape(SGS, QT)
            gcol = sg * SGS + lax.broadcasted_iota(jnp.int32, (SGS, QT), 0)
            cand = jnp.where(dsg == bmb, gcol, big_i)
            pos = jnp.min(cand.reshape(SGS // BS, BS, QT), axis=1)
            posb = jnp.broadcast_to(pos[:, None, :], (SGS // BS, BS, QT))
            posb = posb.reshape(SGS, QT)
            d_ref[sl, :] = jnp.where(gcol == posb, inf, dsg)
            hsl = pl.ds(r * NB + sg * (SGS // BS), SGS // BS)
            hv_ref[hsl, :] = bm
            hi_ref[hsl, :] = pos
            return ()

        lax.fori_loop(0, NSG, head_sg, ())
        return ()

    lax.fori_loop(0, HEADS, head_round, ())

    row16 = lax.broadcasted_iota(jnp.int32, (KNN_K, QT), 0)
    hrank = lax.broadcasted_iota(jnp.int32, (HEADS * NB, QT), 0) // NB

    def sel_pass(k, carry):
        idx_acc, exfl = carry
        hv = hv_ref[...]
        hi = hi_ref[...]
        m = jnp.min(hv, axis=0, keepdims=True)
        sel = hv == m
        cand = jnp.where(sel, hi, big_i)
        amin = jnp.min(cand, axis=0, keepdims=True)
        taken = hi == amin
        idx_acc = jnp.where(row16 == k, jnp.broadcast_to(amin, (KNN_K, QT)),
                            idx_acc)
        selr = jnp.max(jnp.where(taken & sel, hrank, -1), axis=0,
                       keepdims=True)
        exfl = jnp.maximum(exfl, selr)
        hv_ref[...] = jnp.where(taken, inf, hv)
        return idx_acc, exfl

    idx0 = jnp.zeros((KNN_K, QT), jnp.int32)
    ex0 = jnp.full((1, QT), -1, jnp.int32)
    idx_acc, exfl = lax.fori_loop(0, KNN_K, sel_pass, (idx0, ex0))
    idx_ref[...] = idx_acc

    @pl.when(jnp.max(exfl) >= HEADS - 1)
    def _():
        def rebuild_sg(sg, _):
            d_ref[pl.ds(sg * SGS, SGS), :] = _knn_dist_slice(p_ref, qt_ref, sg)
            return ()

        lax.fori_loop(0, NSG, rebuild_sg, ())
        gcol = lax.broadcasted_iota(jnp.int32, (CP, QT), 0)

        def fb_pass(k, idx_acc):
            d = d_ref[...]
            m = jnp.min(d, axis=0, keepdims=True)
            cand = jnp.where(d == m, gcol, big_i)
            amin = jnp.min(cand, axis=0, keepdims=True)
            idx_acc = jnp.where(row16 == k,
                                jnp.broadcast_to(amin, (KNN_K, QT)), idx_acc)
            d_ref[...] = jnp.where(gcol == amin, inf, d)
            return idx_acc

        idx_fb = lax.fori_loop(0, KNN_K, fb_pass, idx0)
        idx_ref[...] = idx_fb


def _knn(pos_pad, pos_t_pad):
    return pl.pallas_call(
        _knn_body,
        grid=(CP // QT,),
        in_specs=[
            pl.BlockSpec((CP, 3), lambda i: (0, 0)),
            pl.BlockSpec((3, QT), lambda i: (0, i)),
        ],
        out_specs=pl.BlockSpec((KNN_K, QT), lambda i: (0, i)),
        out_shape=jax.ShapeDtypeStruct((KNN_K, CP), jnp.int32),
        scratch_shapes=[
            pltpu.VMEM((CP, QT), jnp.float32),
            pltpu.VMEM((HEADS * NB, QT), jnp.float32),
            pltpu.VMEM((HEADS * NB, QT), jnp.int32),
        ],
    )(pos_pad, pos_t_pad)


def _make_sc_gather(c):
    mesh = plsc.VectorSubcoreMesh(core_axis_name="c", subcore_axis_name="s")

    gp2 = 32
    nch2 = BPW // gp2
    nxfer = gp2 * KNN_K // 128

    @functools.partial(
        pl.kernel,
        mesh=mesh,
        compiler_params=pltpu.CompilerParams(use_tc_tiling_on_sc=False),
        out_type=jax.ShapeDtypeStruct((NPAD * KNN_K, c), jnp.float32),
        scratch_types=[
            pltpu.VMEM((BPW * KNN_K,), jnp.int32),
            pltpu.VMEM((gp2 * KNN_K, c), jnp.float32),
            pltpu.SemaphoreType.DMA,
        ],
    )
    def kern(x_hbm, idxf_hbm, out_hbm, idx_v, rows_v, sem):
        wid = lax.axis_index("s") * SC_CORES + lax.axis_index("c")
        base_w = wid * BPW * KNN_K
        pltpu.sync_copy(idxf_hbm.at[pl.ds(base_w, BPW * KNN_K)], idx_v)

        def chunk_body(ci, _):
            base = ci * gp2 * KNN_K
            cps = [
                pltpu.async_copy(
                    x_hbm.at[idx_v.at[pl.ds(base + s * 128, 128)]],
                    rows_v.at[pl.ds(s * 128, 128)], sem)
                for s in range(nxfer)
            ]
            for cp in cps:
                cp.wait()
            pltpu.sync_copy(rows_v, out_hbm.at[pl.ds(base_w + base, gp2 * KNN_K)])
            return ()

        lax.fori_loop(0, nch2, chunk_body, ())

    return kern


def _econv_body(x_ref, xg_ref, wa_ref, wb_ref, b_ref,
                mx_ref, sh_ref, sh2_ref):
    i = pl.program_id(0)
    c = x_ref.shape[1]
    d = wa_ref.shape[1]
    xi = x_ref[...]
    xi3 = jnp.broadcast_to(xi[:, None, :], (ECHUNK, KNN_K, c))
    dvec = xg_ref[...] - xi3.reshape(ECHUNK * KNN_K, c)
    p = jnp.dot(xi, wa_ref[...], preferred_element_type=jnp.float32)
    e = jnp.dot(dvec, wb_ref[...], preferred_element_type=jnp.float32)
    h = (e.reshape(ECHUNK, KNN_K, d) + p[:, None, :]) + b_ref[...][None]
    mx_ref[...] = jnp.max(h, axis=1)
    node = i * ECHUNK + lax.broadcasted_iota(jnp.int32, (ECHUNK, KNN_K, d), 0)
    hv = jnp.where(node < NPTS, h, 0.0)
    psum = jnp.sum(hv, axis=(0, 1), keepdims=False)[None]
    psum2 = jnp.sum(hv * hv, axis=(0, 1), keepdims=False)[None]

    @pl.when(i == 0)
    def _():
        sh_ref[...] = jnp.zeros_like(sh_ref)
        sh2_ref[...] = jnp.zeros_like(sh2_ref)

    sh_ref[...] += psum
    sh2_ref[...] += psum2


def _econv(x_pad, xgather, wa, wb, b):
    c = x_pad.shape[1]
    d = wa.shape[1]
    return pl.pallas_call(
        _econv_body,
        grid=(NPAD // ECHUNK,),
        in_specs=[
            pl.BlockSpec((ECHUNK, c), lambda i: (i, 0)),
            pl.BlockSpec((ECHUNK * KNN_K, c), lambda i: (i, 0)),
            pl.BlockSpec((c, d), lambda i: (0, 0)),
            pl.BlockSpec((c, d), lambda i: (0, 0)),
            pl.BlockSpec((1, d), lambda i: (0, 0)),
        ],
        out_specs=[
            pl.BlockSpec((ECHUNK, d), lambda i: (i, 0)),
            pl.BlockSpec((1, d), lambda i: (0, 0)),
            pl.BlockSpec((1, d), lambda i: (0, 0)),
        ],
        out_shape=[
            jax.ShapeDtypeStruct((NPAD, d), jnp.float32),
            jax.ShapeDtypeStruct((1, d), jnp.float32),
            jax.ShapeDtypeStruct((1, d), jnp.float32),
        ],
    )(x_pad, xgather, wa, wb, b)


def _combine_body(mx_ref, sh_ref, sh2_ref, g_ref, be_ref, o_ref):
    cnt = jnp.float32(NPTS * KNN_K)
    mu = sh_ref[...] / cnt
    ex2 = sh2_ref[...] / cnt
    var = ex2 - mu * mu
    scale = g_ref[...] * lax.rsqrt(var + BN_EPS)
    o_ref[...] = jnp.maximum((mx_ref[...] - mu) * scale + be_ref[...], 0.0)


def _combine(maxh, sh, sh2, g, be, rows_chunk=2000):
    n, d = maxh.shape
    return pl.pallas_call(
        _combine_body,
        grid=(n // rows_chunk,),
        in_specs=[
            pl.BlockSpec((rows_chunk, d), lambda i: (i, 0)),
            pl.BlockSpec((1, d), lambda i: (0, 0)),
            pl.BlockSpec((1, d), lambda i: (0, 0)),
            pl.BlockSpec((1, d), lambda i: (0, 0)),
            pl.BlockSpec((1, d), lambda i: (0, 0)),
        ],
        out_specs=pl.BlockSpec((rows_chunk, d), lambda i: (i, 0)),
        out_shape=jax.ShapeDtypeStruct((n, d), jnp.float32),
    )(maxh, sh, sh2, g, be)


def _edge_conv_layer(x, idx_flat, w, b, g, be, sc_kern, cpad):
    c = x.shape[1]
    d = w.shape[1]
    wa = jnp.zeros((cpad, d), jnp.float32).at[:c].set(w[:c])
    wb = jnp.zeros((cpad, d), jnp.float32).at[:c].set(w[c:])
    x_pad = jnp.zeros((NPAD, cpad), jnp.float32).at[:NPTS, :c].set(x)
    xgather = sc_kern(x_pad, idx_flat)
    maxh, sh, sh2 = _econv(x_pad, xgather, wa, wb, b[None])
    return _combine(maxh[:NPTS], sh, sh2, g[None], be[None])


def kernel(points, W0, b0, g0, be0, W1, b1, g1, be1):
    pos = points[:, 1:4]
    pos_pad = jnp.full((CP, 3), PAD_COORD, jnp.float32).at[:NPTS].set(pos)
    idx_t = _knn(pos_pad, pos_pad.T)
    idx = idx_t[:, :NPTS].T
    idx_pad = jnp.zeros((NPAD, KNN_K), jnp.int32).at[:NPTS].set(idx)
    idx_flat = idx_pad.reshape(-1)

    feat = points[:, 1:]
    sc16 = _make_sc_gather(16)
    sc32 = _make_sc_gather(32)
    feat = _edge_conv_layer(feat, idx_flat, W0, b0, g0, be0, sc16, 16)
    feat = _edge_conv_layer(feat, idx_flat, W1, b1, g1, be1, sc32, 32)
    return feat


# speedup vs baseline: 6.3345x; 1.0972x over previous
import functools

import jax
import jax.numpy as jnp
from jax import lax
from jax.experimental import pallas as pl
from jax.experimental.pallas import tpu as pltpu
from jax.experimental.pallas import tpu_sc as plsc

NPTS = 10000
KNN_K = 16
BN_EPS = 1e-5

SC_CORES = 2
SC_SUBCORES = 16
NWORK = SC_CORES * SC_SUBCORES
NPAD = 10240
BPW = NPAD // NWORK
GP = 8
NCHUNK = BPW // GP

ECHUNK = 128

QT = 128
CP = 10240
BS = 128
NB = CP // BS
SGS = 1024
NSG = CP // SGS
HEADS = 5
PAD_COORD = 1e9


def _knn_dist_slice(p_ref, qt_ref, sg):
    sl = pl.ds(sg * SGS, SGS)
    px = p_ref[sl, 0:1]
    py = p_ref[sl, 1:2]
    pz = p_ref[sl, 2:3]
    qx = qt_ref[0:1, :]
    qy = qt_ref[1:2, :]
    qz = qt_ref[2:3, :]
    dx = px - qx
    dy = py - qy
    dz = pz - qz
    return dx * dx + dy * dy + dz * dz


def _knn_body(p_ref, qt_ref, idx_ref, d_ref, hv_ref, hi_ref):
    inf = jnp.float32(jnp.inf)
    big_i = jnp.int32(1 << 30)

    def build_sg(sg, _):
        d_ref[pl.ds(sg * SGS, SGS), :] = _knn_dist_slice(p_ref, qt_ref, sg)
        return ()

    lax.fori_loop(0, NSG, build_sg, ())

    def head_round(r, _):
        def head_sg(sg, _):
            sl = pl.ds(sg * SGS, SGS)
            dsg = d_ref[sl, :]
            v3 = dsg.reshape(SGS // BS, BS, QT)
            bm = jnp.min(v3, axis=1)
            bmb = jnp.broadcast_to(bm[:, None, :], (SGS // BS, BS, QT))
            bmb = bmb.reshape(SGS, QT)
            gcol = sg * SGS + lax.broadcasted_iota(jnp.int32, (SGS, QT), 0)
            cand = jnp.where(dsg == bmb, gcol, big_i)
            pos = jnp.min(cand.reshape(SGS // BS, BS, QT), axis=1)
            posb = jnp.broadcast_to(pos[:, None, :], (SGS // BS, BS, QT))
            posb = posb.reshape(SGS, QT)
            d_ref[sl, :] = jnp.where(gcol == posb, inf, dsg)
            hsl = pl.ds(r * NB + sg * (SGS // BS), SGS // BS)
            hv_ref[hsl, :] = bm
            hi_ref[hsl, :] = pos
            return ()

        lax.fori_loop(0, NSG, head_sg, ())
        return ()

    lax.fori_loop(0, HEADS, head_round, ())

    row16 = lax.broadcasted_iota(jnp.int32, (KNN_K, QT), 0)
    hrank = lax.broadcasted_iota(jnp.int32, (HEADS * NB, QT), 0) // NB

    def sel_pass(k, carry):
        idx_acc, exfl = carry
        hv = hv_ref[...]
        hi = hi_ref[...]
        m = jnp.min(hv, axis=0, keepdims=True)
        sel = hv == m
        cand = jnp.where(sel, hi, big_i)
        amin = jnp.min(cand, axis=0, keepdims=True)
        taken = hi == amin
        idx_acc = jnp.where(row16 == k, jnp.broadcast_to(amin, (KNN_K, QT)),
                            idx_acc)
        selr = jnp.max(jnp.where(taken & sel, hrank, -1), axis=0,
                       keepdims=True)
        exfl = jnp.maximum(exfl, selr)
        hv_ref[...] = jnp.where(taken, inf, hv)
        return idx_acc, exfl

    idx0 = jnp.zeros((KNN_K, QT), jnp.int32)
    ex0 = jnp.full((1, QT), -1, jnp.int32)
    idx_acc, exfl = lax.fori_loop(0, KNN_K, sel_pass, (idx0, ex0))
    idx_ref[...] = idx_acc

    @pl.when(jnp.max(exfl) >= HEADS - 1)
    def _():
        def rebuild_sg(sg, _):
            d_ref[pl.ds(sg * SGS, SGS), :] = _knn_dist_slice(p_ref, qt_ref, sg)
            return ()

        lax.fori_loop(0, NSG, rebuild_sg, ())
        gcol = lax.broadcasted_iota(jnp.int32, (CP, QT), 0)

        def fb_pass(k, idx_acc):
            d = d_ref[...]
            m = jnp.min(d, axis=0, keepdims=True)
            cand = jnp.where(d == m, gcol, big_i)
            amin = jnp.min(cand, axis=0, keepdims=True)
            idx_acc = jnp.where(row16 == k,
                                jnp.broadcast_to(amin, (KNN_K, QT)), idx_acc)
            d_ref[...] = jnp.where(gcol == amin, inf, d)
            return idx_acc

        idx_fb = lax.fori_loop(0, KNN_K, fb_pass, idx0)
        idx_ref[...] = idx_fb


def _knn(pos_pad, pos_t_pad):
    return pl.pallas_call(
        _knn_body,
        grid=(CP // QT,),
        in_specs=[
            pl.BlockSpec((CP, 3), lambda i: (0, 0)),
            pl.BlockSpec((3, QT), lambda i: (0, i)),
        ],
        out_specs=pl.BlockSpec((KNN_K, QT), lambda i: (0, i)),
        out_shape=jax.ShapeDtypeStruct((KNN_K, CP), jnp.int32),
        scratch_shapes=[
            pltpu.VMEM((CP, QT), jnp.float32),
            pltpu.VMEM((HEADS * NB, QT), jnp.float32),
            pltpu.VMEM((HEADS * NB, QT), jnp.int32),
        ],
    )(pos_pad, pos_t_pad)


def _make_sc_gather(c):
    mesh = plsc.VectorSubcoreMesh(core_axis_name="c", subcore_axis_name="s")

    gp2 = 32
    nch2 = BPW // gp2
    nxfer = gp2 * KNN_K // 128

    @functools.partial(
        pl.kernel,
        mesh=mesh,
        compiler_params=pltpu.CompilerParams(use_tc_tiling_on_sc=False),
        out_type=jax.ShapeDtypeStruct((NPAD * KNN_K, c), jnp.float32),
        scratch_types=[
            pltpu.VMEM((BPW * KNN_K,), jnp.int32),
            pltpu.VMEM((gp2 * KNN_K, c), jnp.float32),
            pltpu.SemaphoreType.DMA,
        ],
    )
    def kern(x_hbm, idxf_hbm, out_hbm, idx_v, rows_v, sem):
        wid = lax.axis_index("s") * SC_CORES + lax.axis_index("c")
        base_w = wid * BPW * KNN_K
        pltpu.sync_copy(idxf_hbm.at[pl.ds(base_w, BPW * KNN_K)], idx_v)

        def chunk_body(ci, _):
            base = ci * gp2 * KNN_K
            cps = [
                pltpu.async_copy(
                    x_hbm.at[idx_v.at[pl.ds(base + s * 128, 128)]],
                    rows_v.at[pl.ds(s * 128, 128)], sem)
                for s in range(nxfer)
            ]
            for cp in cps:
                cp.wait()
            pltpu.sync_copy(rows_v, out_hbm.at[pl.ds(base_w + base, gp2 * KNN_K)])
            return ()

        lax.fori_loop(0, nch2, chunk_body, ())

    return kern


def _econv_body(x_ref, xg_ref, wa_ref, wb_ref, b_ref,
                mx_ref, sh_ref, sh2_ref):
    i = pl.program_id(0)
    c = x_ref.shape[1]
    d = wa_ref.shape[1]
    xi = x_ref[...]
    xi3 = jnp.broadcast_to(xi[:, None, :], (ECHUNK, KNN_K, c))
    dvec = xg_ref[...] - xi3.reshape(ECHUNK * KNN_K, c)
    p = jnp.dot(xi, wa_ref[...], preferred_element_type=jnp.float32)
    e = jnp.dot(dvec, wb_ref[...], preferred_element_type=jnp.float32)
    h = (e.reshape(ECHUNK, KNN_K, d) + p[:, None, :]) + b_ref[...][None]
    mx_ref[...] = jnp.max(h, axis=1)
    node = i * ECHUNK + lax.broadcasted_iota(jnp.int32, (ECHUNK, KNN_K, d), 0)
    hv = jnp.where(node < NPTS, h, 0.0)
    psum = jnp.sum(hv, axis=(0, 1), keepdims=False)[None]
    psum2 = jnp.sum(hv * hv, axis=(0, 1), keepdims=False)[None]

    @pl.when(i == 0)
    def _():
        sh_ref[...] = jnp.zeros_like(sh_ref)
        sh2_ref[...] = jnp.zeros_like(sh2_ref)

    sh_ref[...] += psum
    sh2_ref[...] += psum2


def _econv(x_pad, xgather, wa, wb, b):
    c = x_pad.shape[1]
    d = wa.shape[1]
    return pl.pallas_call(
        _econv_body,
        grid=(NPAD // ECHUNK,),
        in_specs=[
            pl.BlockSpec((ECHUNK, c), lambda i: (i, 0)),
            pl.BlockSpec((ECHUNK * KNN_K, c), lambda i: (i, 0)),
            pl.BlockSpec((c, d), lambda i: (0, 0)),
            pl.BlockSpec((c, d), lambda i: (0, 0)),
            pl.BlockSpec((1, d), lambda i: (0, 0)),
        ],
        out_specs=[
            pl.BlockSpec((ECHUNK, d), lambda i: (i, 0)),
            pl.BlockSpec((1, d), lambda i: (0, 0)),
            pl.BlockSpec((1, d), lambda i: (0, 0)),
        ],
        out_shape=[
            jax.ShapeDtypeStruct((NPAD, d), jnp.float32),
            jax.ShapeDtypeStruct((1, d), jnp.float32),
            jax.ShapeDtypeStruct((1, d), jnp.float32),
        ],
    )(x_pad, xgather, wa, wb, b)


def _combine_body(mx_ref, sh_ref, sh2_ref, g_ref, be_ref, o_ref):
    cnt = jnp.float32(NPTS * KNN_K)
    mu = sh_ref[...] / cnt
    ex2 = sh2_ref[...] / cnt
    var = ex2 - mu * mu
    scale = g_ref[...] * lax.rsqrt(var + BN_EPS)
    o_ref[...] = jnp.maximum((mx_ref[...] - mu) * scale + be_ref[...], 0.0)


def _combine(maxh, sh, sh2, g, be, rows_chunk=2000):
    n, d = maxh.shape
    return pl.pallas_call(
        _combine_body,
        grid=(n // rows_chunk,),
        in_specs=[
            pl.BlockSpec((rows_chunk, d), lambda i: (i, 0)),
            pl.BlockSpec((1, d), lambda i: (0, 0)),
            pl.BlockSpec((1, d), lambda i: (0, 0)),
            pl.BlockSpec((1, d), lambda i: (0, 0)),
            pl.BlockSpec((1, d), lambda i: (0, 0)),
        ],
        out_specs=pl.BlockSpec((rows_chunk, d), lambda i: (i, 0)),
        out_shape=jax.ShapeDtypeStruct((n, d), jnp.float32),
    )(maxh, sh, sh2, g, be)


def _edge_conv_layer(x, idx_flat, w, b, g, be, sc_kern, cpad):
    c = x.shape[1]
    d = w.shape[1]
    wa = jnp.zeros((cpad, d), jnp.float32).at[:c].set(w[:c])
    wb = jnp.zeros((cpad, d), jnp.float32).at[:c].set(w[c:])
    x_pad = jnp.zeros((NPAD, cpad), jnp.float32).at[:NPTS, :c].set(x)
    xgather = sc_kern(x_pad, idx_flat)
    maxh, sh, sh2 = _econv(x_pad, xgather, wa, wb, b[None])
    return _combine(maxh[:NPTS], sh, sh2, g[None], be[None])


def kernel(points, W0, b0, g0, be0, W1, b1, g1, be1):
    pos = points[:, 1:4]
    pos_pad = jnp.full((CP, 3), PAD_COORD, jnp.float32).at[:NPTS].set(pos)
    idx_t = _knn(pos_pad, pos_pad.T)
    idx = idx_t[:, :NPTS].T
    idx_pad = jnp.zeros((NPAD, KNN_K), jnp.int32).at[:NPTS].set(idx)
    idx_flat = idx_pad.reshape(-1)

    feat = points[:, 1:]
    sc16 = _make_sc_gather(16)
    sc32 = _make_sc_gather(32)
    feat = _edge_conv_layer(feat, idx_flat, W0, b0, g0, be0, sc16, 16)
    feat = _edge_conv_layer(feat, idx_flat, W1, b1, g1, be1, sc32, 32)
    return feat
